# baseline (speedup 1.0000x reference)
import jax
import jax.numpy as jnp
from jax.experimental import pallas as pl
from jax.experimental.pallas import tpu as pltpu

_G = 16


def _pool_kernel(tok_ref, perm_ref, cnt_ref, lenf_ref, emb_hbm, out_ref,
                 emb_vmem, sem):
    c = pl.program_id(0)
    j = pl.program_id(1)
    nj = pl.num_programs(1)
    tb, _, D = out_ref.shape
    seq_len = tok_ref.shape[1]
    half = seq_len // 2
    G = _G
    gshift = G.bit_length() - 1

    @pl.when(j == 0)
    def _():
        cp = pltpu.make_async_copy(emb_hbm, emb_vmem.at[:, 0], sem)
        cp.start()
        cp.wait()

    t = c * nj + j
    base = t * tb

    def make_group_body(l0, l1, first):
        def group_body(g, carry):
            k0 = base + g * G
            locs = [perm_ref[k0 + i] for i in range(G)]
            rows = [base + locs[i] for i in range(G)]
            accs = [emb_vmem[tok_ref[rows[i], l0]] for i in range(G)]
            for l in range(l0 + 1, l1):
                for i in range(G):
                    accs[i] = accs[i] + emb_vmem[tok_ref[rows[i], l]]
            scaled = [accs[i] / lenf_ref[rows[i]] for i in range(G)]
            if first:
                for i in range(G):
                    out_ref[locs[i]] = scaled[i]
            else:
                prev = [out_ref[locs[i]] for i in range(G)]
                for i in range(G):
                    out_ref[locs[i]] = prev[i] + scaled[i]
            return carry

        return group_body

    jax.lax.fori_loop(0, tb // G, make_group_body(0, half, True), 0)
    trips = jax.lax.shift_right_logical(cnt_ref[t, 0] + (G - 1), gshift)
    jax.lax.fori_loop(0, trips, make_group_body(half, seq_len, False), 0)


def kernel(tok_batch, tok_lens, emb_table):
    B, L = tok_batch.shape
    V, D = emb_table.shape

    n_cores = 2
    tb = 128
    if B % (n_cores * tb) != 0:
        tb = B // n_cores
    tiles_per_core = B // (n_cores * tb)
    n_tiles = n_cores * tiles_per_core

    tok_i32 = tok_batch.astype(jnp.int32)
    lens_i32 = tok_lens.astype(jnp.int32)
    lens_f32 = tok_lens.astype(jnp.float32)
    emb2 = emb_table.astype(jnp.float32)

    lens2d = lens_i32.reshape(n_tiles, tb)
    perm = jnp.argsort(-lens2d, axis=1).astype(jnp.int32)
    cnts = jnp.sum(lens2d > (L // 2), axis=1, dtype=jnp.int32).reshape(
        n_tiles, 1)
    perm_flat = perm.reshape(B)

    grid_spec = pltpu.PrefetchScalarGridSpec(
        num_scalar_prefetch=4,
        grid=(n_cores, tiles_per_core),
        in_specs=[pl.BlockSpec(memory_space=pl.ANY)],
        out_specs=pl.BlockSpec(
            (tb, 1, D), lambda c, j, tok, pm, ct, lf: (c * tiles_per_core + j, 0, 0)
        ),
        scratch_shapes=[
            pltpu.VMEM((V, 1, D), jnp.float32),
            pltpu.SemaphoreType.DMA,
        ],
    )

    out = pl.pallas_call(
        _pool_kernel,
        out_shape=jax.ShapeDtypeStruct((B, 1, D), jnp.float32),
        grid_spec=grid_spec,
        compiler_params=pltpu.CompilerParams(
            dimension_semantics=("parallel", "arbitrary"),
            vmem_limit_bytes=44 << 20,
        ),
    )(tok_i32, perm_flat, cnts, lens_f32, emb2)
    return out.reshape(B, D)


# speedup vs baseline: 1.0801x; 1.0801x over previous
import jax
import jax.numpy as jnp
from jax.experimental import pallas as pl
from jax.experimental.pallas import tpu as pltpu


def _pool_kernel(tok_ref, lenf_ref, emb_hbm, out_ref, emb_vmem, sem):
    c = pl.program_id(0)
    j = pl.program_id(1)
    nj = pl.num_programs(1)
    tb, _, D = out_ref.shape
    seq_len = tok_ref.shape[1]

    @pl.when(j == 0)
    def _():
        cp = pltpu.make_async_copy(emb_hbm, emb_vmem.at[:, 0], sem)
        cp.start()
        cp.wait()

    base = (c * nj + j) * tb

    G = 16

    def group_body(g, carry):
        b0 = base + g * G
        rows = [b0 + i for i in range(G)]
        accs = [emb_vmem[tok_ref[rows[i], 0]] for i in range(G)]
        for l in range(1, seq_len):
            for i in range(G):
                accs[i] = accs[i] + emb_vmem[tok_ref[rows[i], l]]
        for i in range(G):
            out_ref[g * G + i] = accs[i] / lenf_ref[rows[i]]
        return carry

    jax.lax.fori_loop(0, tb // G, group_body, 0)


def kernel(tok_batch, tok_lens, emb_table):
    B, L = tok_batch.shape
    V, D = emb_table.shape

    n_cores = 2
    tb = 128
    if B % (n_cores * tb) != 0:
        tb = B // n_cores
    tiles_per_core = B // (n_cores * tb)

    tok_i32 = tok_batch.astype(jnp.int32)
    lens_f32 = tok_lens.astype(jnp.float32)
    emb2 = emb_table.astype(jnp.float32)

    grid_spec = pltpu.PrefetchScalarGridSpec(
        num_scalar_prefetch=2,
        grid=(n_cores, tiles_per_core),
        in_specs=[pl.BlockSpec(memory_space=pl.ANY)],
        out_specs=pl.BlockSpec(
            (tb, 1, D), lambda c, j, tok, lf: (c * tiles_per_core + j, 0, 0)
        ),
        scratch_shapes=[
            pltpu.VMEM((V, 1, D), jnp.float32),
            pltpu.SemaphoreType.DMA,
        ],
    )

    out = pl.pallas_call(
        _pool_kernel,
        out_shape=jax.ShapeDtypeStruct((B, 1, D), jnp.float32),
        grid_spec=grid_spec,
        compiler_params=pltpu.CompilerParams(
            dimension_semantics=("parallel", "arbitrary"),
            vmem_limit_bytes=44 << 20,
        ),
    )(tok_i32, lens_f32, emb2)
    return out.reshape(B, D)
